# baseline (speedup 1.0000x reference)
import functools

import jax
import jax.numpy as jnp
from jax import lax
from jax.experimental import pallas as pl
from jax.experimental.pallas import tpu as pltpu
from jax.experimental.pallas import tpu_sc as plsc

_CLASSES = 1000
_EPS = 1e-12
_BINS = 1024
_N = 16384
_ROWS_PER_BLOCK = 2048
_NUM_BLOCKS = _N // _ROWS_PER_BLOCK

_NUM_WORKERS = 32
_CHUNK = _N // _NUM_WORKERS
_ITERS = _CHUNK // 16


def _argmax_body(x_ref, out_ref):
    x = x_ref[...]
    m = jnp.max(x, axis=1, keepdims=True)
    col = lax.broadcasted_iota(jnp.int32, x.shape, 1)
    out_ref[...] = jnp.min(jnp.where(x == m, col, _CLASSES), axis=1)


_NTILES = 16
_SC_CHUNK = _N // _NTILES
_SC_ITERS = _SC_CHUNK // 16
_HSZ = 3 * _BINS


@functools.cache
def _make_sc_hist_f1():
    mesh = plsc.VectorSubcoreMesh(core_axis_name="c", subcore_axis_name="s")

    @functools.partial(
        pl.kernel,
        mesh=mesh,
        compiler_params=pltpu.CompilerParams(needs_layout_passes=False),
        out_type=(
            jax.ShapeDtypeStruct((_NTILES * _HSZ,), jnp.float32),
            jax.ShapeDtypeStruct((16,), jnp.float32),
        ),
        scratch_types=[
            pltpu.VMEM((_SC_CHUNK,), jnp.int32),
            pltpu.VMEM((_SC_CHUNK,), jnp.int32),
            pltpu.VMEM((_BINS,), jnp.float32),
            pltpu.VMEM((_BINS,), jnp.float32),
            pltpu.VMEM((_BINS,), jnp.float32),
            pltpu.VMEM((_HSZ,), jnp.float32),
            pltpu.VMEM((16,), jnp.float32),
        ],
    )
    def _sc_hist_f1(preds_hbm, ytrue_hbm, part_hbm, res_hbm,
                    p_v, t_v, h_true, h_pred, h_tp, tmp_v, res_v):
        cid = lax.axis_index("c")
        sid = lax.axis_index("s")

        @pl.when(cid == 0)
        def _work():
            base = sid * _SC_CHUNK
            pltpu.sync_copy(preds_hbm.at[pl.ds(base, _SC_CHUNK)], p_v)
            pltpu.sync_copy(ytrue_hbm.at[pl.ds(base, _SC_CHUNK)], t_v)
            zeros = jnp.zeros((16,), jnp.float32)
            for j in range(_BINS // 16):
                s = pl.ds(j * 16, 16)
                h_true[s] = zeros
                h_pred[s] = zeros
                h_tp[s] = zeros
            ones = jnp.full((16,), 1.0, jnp.float32)
            for i in range(_SC_ITERS):
                s = pl.ds(i * 16, 16)
                p = p_v[s]
                t = t_v[s]
                plsc.addupdate_scatter(h_pred, [p], ones)
                plsc.addupdate_scatter(h_true, [t], ones)
                plsc.addupdate_scatter(h_tp, [t], ones, mask=p == t)
            obase = sid * _HSZ
            pltpu.sync_copy(h_true, part_hbm.at[pl.ds(obase, _BINS)])
            pltpu.sync_copy(h_pred, part_hbm.at[pl.ds(obase + _BINS, _BINS)])
            pltpu.sync_copy(h_tp, part_hbm.at[pl.ds(obase + 2 * _BINS, _BINS)])
            plsc.subcore_barrier()

            @pl.when(sid == 0)
            def _finalize():
                def _acc_one(r, carry):
                    pltpu.sync_copy(part_hbm.at[pl.ds(r * _HSZ, _HSZ)], tmp_v)
                    for j in range(_BINS // 16):
                        s = pl.ds(j * 16, 16)
                        h_true[s] = h_true[s] + tmp_v[pl.ds(j * 16, 16)]
                        h_pred[s] = h_pred[s] + tmp_v[pl.ds(_BINS + j * 16, 16)]
                        h_tp[s] = h_tp[s] + tmp_v[pl.ds(2 * _BINS + j * 16, 16)]
                    return carry

                lax.fori_loop(1, _NTILES, _acc_one, 0)
                acc = jnp.zeros((16,), jnp.float32)
                for j in range(_BINS // 16):
                    s = pl.ds(j * 16, 16)
                    ct = h_true[s]
                    cp = h_pred[s]
                    tp = h_tp[s]
                    sens = tp / (cp + _EPS)
                    prec = tp / (ct + _EPS)
                    acc = acc + 2.0 * (prec * sens) / (prec + sens + _EPS)
                total = jnp.sum(acc * jnp.float32(1.0 / _CLASSES))
                res_v[...] = jnp.full((16,), total, jnp.float32)
                pltpu.sync_copy(res_v, res_hbm)

    return _sc_hist_f1


@jax.jit
def kernel(y_pred, y_true):
    preds = pl.pallas_call(
        _argmax_body,
        grid=(_NUM_BLOCKS,),
        in_specs=[pl.BlockSpec((_ROWS_PER_BLOCK, _CLASSES), lambda i: (i, 0))],
        out_specs=pl.BlockSpec((_ROWS_PER_BLOCK,), lambda i: (i,)),
        out_shape=jax.ShapeDtypeStruct((_N,), jnp.int32),
    )(y_pred)
    _, res = _make_sc_hist_f1()(preds, y_true)
    return res[0]


# speedup vs baseline: 1.0737x; 1.0737x over previous
import functools

import jax
import jax.numpy as jnp
from jax import lax
from jax.experimental import pallas as pl
from jax.experimental.pallas import tpu as pltpu
from jax.experimental.pallas import tpu_sc as plsc

_CLASSES = 1000
_EPS = 1e-12
_BINS = 1024
_N = 16384
_ROWS_PER_BLOCK = 2048
_NUM_BLOCKS = _N // _ROWS_PER_BLOCK

_NUM_WORKERS = 32
_CHUNK = _N // _NUM_WORKERS
_ITERS = _CHUNK // 16


def _argmax_body(x_ref, out_ref):
    x = x_ref[...]
    m = jnp.max(x, axis=1, keepdims=True)
    col = lax.broadcasted_iota(jnp.int32, x.shape, 1)
    out_ref[...] = jnp.min(jnp.where(x == m, col, _CLASSES), axis=1)


_NTILES = 16
_SC_CHUNK = _N // _NTILES
_SC_ITERS = _SC_CHUNK // 16
_HSZ = 3 * _BINS


@functools.cache
def _make_sc_hist_f1():
    mesh = plsc.VectorSubcoreMesh(
        core_axis_name="c", subcore_axis_name="s", num_cores=1
    )

    @functools.partial(
        pl.kernel,
        mesh=mesh,
        compiler_params=pltpu.CompilerParams(needs_layout_passes=False),
        out_type=jax.ShapeDtypeStruct((16,), jnp.float32),
        scratch_types=[
            pltpu.VMEM((_SC_CHUNK,), jnp.int32),
            pltpu.VMEM((_SC_CHUNK,), jnp.int32),
            pltpu.VMEM((_BINS,), jnp.float32),
            pltpu.VMEM((_BINS,), jnp.float32),
            pltpu.VMEM((_BINS,), jnp.float32),
            pltpu.VMEM((_HSZ,), jnp.float32),
            pltpu.VMEM((16,), jnp.float32),
            pltpu.VMEM_SHARED((_NTILES * _HSZ,), jnp.float32),
        ],
    )
    def _sc_hist_f1(preds_hbm, ytrue_hbm, res_hbm,
                    p_v, t_v, h_true, h_pred, h_tp, tmp_v, res_v, shared):
        sid = lax.axis_index("s")
        base = sid * _SC_CHUNK
        pltpu.sync_copy(preds_hbm.at[pl.ds(base, _SC_CHUNK)], p_v)
        pltpu.sync_copy(ytrue_hbm.at[pl.ds(base, _SC_CHUNK)], t_v)
        zeros = jnp.zeros((16,), jnp.float32)
        for j in range(_BINS // 16):
            s = pl.ds(j * 16, 16)
            h_true[s] = zeros
            h_pred[s] = zeros
            h_tp[s] = zeros
        ones = jnp.full((16,), 1.0, jnp.float32)
        for i in range(_SC_ITERS):
            s = pl.ds(i * 16, 16)
            p = p_v[s]
            t = t_v[s]
            plsc.addupdate_scatter(h_pred, [p], ones)
            plsc.addupdate_scatter(h_true, [t], ones)
            plsc.addupdate_scatter(h_tp, [t], ones, mask=p == t)
        obase = sid * _HSZ
        pltpu.sync_copy(h_true, shared.at[pl.ds(obase, _BINS)])
        pltpu.sync_copy(h_pred, shared.at[pl.ds(obase + _BINS, _BINS)])
        pltpu.sync_copy(h_tp, shared.at[pl.ds(obase + 2 * _BINS, _BINS)])
        plsc.subcore_barrier()

        @pl.when(sid == 0)
        def _finalize():
            def _acc_one(r, carry):
                pltpu.sync_copy(shared.at[pl.ds(r * _HSZ, _HSZ)], tmp_v)
                for j in range(_BINS // 16):
                    s = pl.ds(j * 16, 16)
                    h_true[s] = h_true[s] + tmp_v[pl.ds(j * 16, 16)]
                    h_pred[s] = h_pred[s] + tmp_v[pl.ds(_BINS + j * 16, 16)]
                    h_tp[s] = h_tp[s] + tmp_v[pl.ds(2 * _BINS + j * 16, 16)]
                return carry

            lax.fori_loop(1, _NTILES, _acc_one, 0)
            acc = jnp.zeros((16,), jnp.float32)
            for j in range(_BINS // 16):
                s = pl.ds(j * 16, 16)
                ct = h_true[s]
                cp = h_pred[s]
                tp = h_tp[s]
                sens = tp / (cp + _EPS)
                prec = tp / (ct + _EPS)
                acc = acc + 2.0 * (prec * sens) / (prec + sens + _EPS)
            total = jnp.sum(acc * jnp.float32(1.0 / _CLASSES))
            res_v[...] = jnp.full((16,), total, jnp.float32)
            pltpu.sync_copy(res_v, res_hbm)

    return _sc_hist_f1


@jax.jit
def kernel(y_pred, y_true):
    preds = pl.pallas_call(
        _argmax_body,
        grid=(_NUM_BLOCKS,),
        in_specs=[pl.BlockSpec((_ROWS_PER_BLOCK, _CLASSES), lambda i: (i, 0))],
        out_specs=pl.BlockSpec((_ROWS_PER_BLOCK,), lambda i: (i,)),
        out_shape=jax.ShapeDtypeStruct((_N,), jnp.int32),
    )(y_pred)
    res = _make_sc_hist_f1()(preds, y_true)
    return res[0]


# speedup vs baseline: 1.1036x; 1.0279x over previous
import functools

import jax
import jax.numpy as jnp
from jax import lax
from jax.experimental import pallas as pl
from jax.experimental.pallas import tpu as pltpu
from jax.experimental.pallas import tpu_sc as plsc

_CLASSES = 1000
_EPS = 1e-12
_BINS = 1024
_N = 16384
_ROWS_PER_BLOCK = 2048
_NUM_BLOCKS = _N // _ROWS_PER_BLOCK

_NUM_WORKERS = 32
_CHUNK = _N // _NUM_WORKERS
_ITERS = _CHUNK // 16


def _argmax_body(x_ref, out_ref):
    x = x_ref[...]
    m = jnp.max(x, axis=1, keepdims=True)
    col = lax.broadcasted_iota(jnp.int32, x.shape, 1)
    out_ref[...] = jnp.min(jnp.where(x == m, col, _CLASSES), axis=1)


_NTILES = 16
_SC_CHUNK = _N // _NTILES
_SC_ITERS = _SC_CHUNK // 16
_HSZ = 3 * _BINS


@functools.cache
def _make_sc_hist_f1():
    mesh = plsc.VectorSubcoreMesh(
        core_axis_name="c", subcore_axis_name="s", num_cores=1
    )

    @functools.partial(
        pl.kernel,
        mesh=mesh,
        compiler_params=pltpu.CompilerParams(needs_layout_passes=False),
        out_type=jax.ShapeDtypeStruct((16,), jnp.float32),
        scratch_types=[
            pltpu.VMEM((_SC_CHUNK,), jnp.int32),
            pltpu.VMEM((_SC_CHUNK,), jnp.int32),
            pltpu.VMEM((_BINS,), jnp.float32),
            pltpu.VMEM((_BINS,), jnp.float32),
            pltpu.VMEM((_BINS,), jnp.float32),
            pltpu.VMEM(((_NTILES - 1) * _HSZ,), jnp.float32),
            pltpu.VMEM((16,), jnp.float32),
            pltpu.VMEM_SHARED((_NTILES * _HSZ,), jnp.float32),
        ],
    )
    def _sc_hist_f1(preds_hbm, ytrue_hbm, res_hbm,
                    p_v, t_v, h_true, h_pred, h_tp, tmp_v, res_v, shared):
        sid = lax.axis_index("s")
        base = sid * _SC_CHUNK
        pltpu.sync_copy(preds_hbm.at[pl.ds(base, _SC_CHUNK)], p_v)
        pltpu.sync_copy(ytrue_hbm.at[pl.ds(base, _SC_CHUNK)], t_v)
        zeros = jnp.zeros((16,), jnp.float32)
        for j in range(_BINS // 16):
            s = pl.ds(j * 16, 16)
            h_true[s] = zeros
            h_pred[s] = zeros
            h_tp[s] = zeros
        ones = jnp.full((16,), 1.0, jnp.float32)
        for i in range(_SC_ITERS):
            s = pl.ds(i * 16, 16)
            p = p_v[s]
            t = t_v[s]
            plsc.addupdate_scatter(h_pred, [p], ones)
            plsc.addupdate_scatter(h_true, [t], ones)
            plsc.addupdate_scatter(h_tp, [t], ones, mask=p == t)
        obase = sid * _HSZ
        pltpu.sync_copy(h_true, shared.at[pl.ds(obase, _BINS)])
        pltpu.sync_copy(h_pred, shared.at[pl.ds(obase + _BINS, _BINS)])
        pltpu.sync_copy(h_tp, shared.at[pl.ds(obase + 2 * _BINS, _BINS)])
        plsc.subcore_barrier()

        @pl.when(sid == 0)
        def _finalize():
            pltpu.sync_copy(shared.at[pl.ds(_HSZ, (_NTILES - 1) * _HSZ)], tmp_v)
            for h_ref, hoff in ((h_true, 0), (h_pred, _BINS), (h_tp, 2 * _BINS)):
                for j in range(_BINS // 16):
                    s = pl.ds(j * 16, 16)
                    a = h_ref[s]
                    for r in range(_NTILES - 1):
                        a = a + tmp_v[pl.ds(r * _HSZ + hoff + j * 16, 16)]
                    h_ref[s] = a
            acc = jnp.zeros((16,), jnp.float32)
            for j in range(_BINS // 16):
                s = pl.ds(j * 16, 16)
                ct = h_true[s]
                cp = h_pred[s]
                tp = h_tp[s]
                sens = tp / (cp + _EPS)
                prec = tp / (ct + _EPS)
                acc = acc + 2.0 * (prec * sens) / (prec + sens + _EPS)
            total = jnp.sum(acc * jnp.float32(1.0 / _CLASSES))
            res_v[...] = jnp.full((16,), total, jnp.float32)
            pltpu.sync_copy(res_v, res_hbm)

    return _sc_hist_f1


@jax.jit
def kernel(y_pred, y_true):
    preds = pl.pallas_call(
        _argmax_body,
        grid=(_NUM_BLOCKS,),
        in_specs=[pl.BlockSpec((_ROWS_PER_BLOCK, _CLASSES), lambda i: (i, 0))],
        out_specs=pl.BlockSpec((_ROWS_PER_BLOCK,), lambda i: (i,)),
        out_shape=jax.ShapeDtypeStruct((_N,), jnp.int32),
    )(y_pred)
    res = _make_sc_hist_f1()(preds, y_true)
    return res[0]


# speedup vs baseline: 1.1782x; 1.0676x over previous
import functools

import jax
import jax.numpy as jnp
from jax import lax
from jax.experimental import pallas as pl
from jax.experimental.pallas import tpu as pltpu
from jax.experimental.pallas import tpu_sc as plsc

_CLASSES = 1000
_EPS = 1e-12
_BINS = 1024
_N = 16384
_ROWS_PER_BLOCK = 2048
_NUM_BLOCKS = _N // _ROWS_PER_BLOCK

_NUM_WORKERS = 32
_CHUNK = _N // _NUM_WORKERS
_ITERS = _CHUNK // 16


def _argmax_body(x_ref, out_ref):
    rows = x_ref.shape[0]
    nfull = _CLASSES // 128
    rem = _CLASSES - nfull * 128
    lane = lax.broadcasted_iota(jnp.int32, (rows, 128), 1)
    best_v = x_ref[:, 0:128]
    best_i = lane
    for c in range(1, nfull):
        v = x_ref[:, c * 128:(c + 1) * 128]
        take = v > best_v
        best_v = jnp.where(take, v, best_v)
        best_i = jnp.where(take, lane + c * 128, best_i)
    tail = x_ref[:, nfull * 128:_CLASSES]
    v = jnp.concatenate(
        [tail, jnp.full((rows, 128 - rem), -jnp.inf, jnp.float32)], axis=1
    )
    take = v > best_v
    best_v = jnp.where(take, v, best_v)
    best_i = jnp.where(take, lane + nfull * 128, best_i)
    m = jnp.max(best_v, axis=1, keepdims=True)
    out_ref[...] = jnp.min(jnp.where(best_v == m, best_i, _CLASSES), axis=1)


@functools.cache
def _make_sc_hist():
    mesh = plsc.VectorSubcoreMesh(core_axis_name="c", subcore_axis_name="s")

    @functools.partial(
        pl.kernel,
        mesh=mesh,
        compiler_params=pltpu.CompilerParams(needs_layout_passes=False),
        out_type=jax.ShapeDtypeStruct((_NUM_WORKERS * 3 * _BINS,), jnp.float32),
        scratch_types=[
            pltpu.VMEM((_CHUNK,), jnp.int32),
            pltpu.VMEM((_CHUNK,), jnp.int32),
            pltpu.VMEM((_BINS,), jnp.float32),
            pltpu.VMEM((_BINS,), jnp.float32),
            pltpu.VMEM((_BINS,), jnp.float32),
        ],
    )
    def _sc_hist(preds_hbm, ytrue_hbm, out_hbm, p_v, t_v, h_true, h_pred, h_tp):
        wid = lax.axis_index("s") * 2 + lax.axis_index("c")
        base = wid * _CHUNK
        pltpu.sync_copy(preds_hbm.at[pl.ds(base, _CHUNK)], p_v)
        pltpu.sync_copy(ytrue_hbm.at[pl.ds(base, _CHUNK)], t_v)
        zeros = jnp.zeros((16,), jnp.float32)
        for j in range(_BINS // 16):
            s = pl.ds(j * 16, 16)
            h_true[s] = zeros
            h_pred[s] = zeros
            h_tp[s] = zeros
        ones = jnp.full((16,), 1.0, jnp.float32)
        for i in range(_ITERS):
            s = pl.ds(i * 16, 16)
            p = p_v[s]
            t = t_v[s]
            plsc.addupdate_scatter(h_pred, [p], ones)
            plsc.addupdate_scatter(h_true, [t], ones)
            plsc.addupdate_scatter(h_tp, [t], ones, mask=p == t)
        obase = wid * 3 * _BINS
        pltpu.sync_copy(h_true, out_hbm.at[pl.ds(obase, _BINS)])
        pltpu.sync_copy(h_pred, out_hbm.at[pl.ds(obase + _BINS, _BINS)])
        pltpu.sync_copy(h_tp, out_hbm.at[pl.ds(obase + 2 * _BINS, _BINS)])

    return _sc_hist


def _f1_body(h_ref, out_ref):
    hs = jnp.sum(h_ref[...], axis=0)
    ct = hs[0:1, :]
    cp = hs[1:2, :]
    tp = hs[2:3, :]
    sens = tp / (cp + _EPS)
    prec = tp / (ct + _EPS)
    f1 = 2.0 * (prec * sens) / (prec + sens + _EPS)
    out_ref[0, 0] = jnp.sum(f1) / _CLASSES


@jax.jit
def kernel(y_pred, y_true):
    preds = pl.pallas_call(
        _argmax_body,
        grid=(_NUM_BLOCKS,),
        in_specs=[pl.BlockSpec((_ROWS_PER_BLOCK, _CLASSES), lambda i: (i, 0))],
        out_specs=pl.BlockSpec((_ROWS_PER_BLOCK,), lambda i: (i,)),
        out_shape=jax.ShapeDtypeStruct((_N,), jnp.int32),
    )(y_pred)
    partials = _make_sc_hist()(preds, y_true).reshape(_NUM_WORKERS, 3, _BINS)
    res = pl.pallas_call(
        _f1_body,
        out_shape=jax.ShapeDtypeStruct((1, 1), jnp.float32),
        out_specs=pl.BlockSpec(memory_space=pltpu.SMEM),
    )(partials)
    return res[0, 0]


# speedup vs baseline: 1.1832x; 1.0042x over previous
import functools

import jax
import jax.numpy as jnp
from jax import lax
from jax.experimental import pallas as pl
from jax.experimental.pallas import tpu as pltpu
from jax.experimental.pallas import tpu_sc as plsc

_CLASSES = 1000
_EPS = 1e-12
_BINS = 1024
_N = 16384
_ROWS_PER_BLOCK = 2048
_NUM_BLOCKS = _N // _ROWS_PER_BLOCK

_NUM_WORKERS = 32
_CHUNK = _N // _NUM_WORKERS
_ITERS = _CHUNK // 16


def _argmax_body(x_ref, out_ref):
    x = x_ref[...]
    m = jnp.max(x, axis=1, keepdims=True)
    col = lax.broadcasted_iota(jnp.int32, x.shape, 1)
    out_ref[...] = jnp.min(jnp.where(x == m, col, _CLASSES), axis=1)


@functools.cache
def _make_sc_hist():
    mesh = plsc.VectorSubcoreMesh(core_axis_name="c", subcore_axis_name="s")

    @functools.partial(
        pl.kernel,
        mesh=mesh,
        compiler_params=pltpu.CompilerParams(needs_layout_passes=False),
        out_type=jax.ShapeDtypeStruct((_NUM_WORKERS * 3 * _BINS,), jnp.float32),
        scratch_types=[
            pltpu.VMEM((_CHUNK,), jnp.int32),
            pltpu.VMEM((_CHUNK,), jnp.int32),
            pltpu.VMEM((_BINS,), jnp.float32),
            pltpu.VMEM((_BINS,), jnp.float32),
            pltpu.VMEM((_BINS,), jnp.float32),
        ],
    )
    def _sc_hist(preds_hbm, ytrue_hbm, out_hbm, p_v, t_v, h_true, h_pred, h_tp):
        wid = lax.axis_index("s") * 2 + lax.axis_index("c")
        base = wid * _CHUNK
        pltpu.sync_copy(preds_hbm.at[pl.ds(base, _CHUNK)], p_v)
        pltpu.sync_copy(ytrue_hbm.at[pl.ds(base, _CHUNK)], t_v)
        zeros = jnp.zeros((16,), jnp.float32)
        for j in range(_BINS // 16):
            s = pl.ds(j * 16, 16)
            h_true[s] = zeros
            h_pred[s] = zeros
            h_tp[s] = zeros
        ones = jnp.full((16,), 1.0, jnp.float32)
        for i in range(_ITERS):
            s = pl.ds(i * 16, 16)
            p = p_v[s]
            t = t_v[s]
            plsc.addupdate_scatter(h_pred, [p], ones)
            plsc.addupdate_scatter(h_true, [t], ones)
            plsc.addupdate_scatter(h_tp, [t], ones, mask=p == t)
        obase = wid * 3 * _BINS
        pltpu.sync_copy(h_true, out_hbm.at[pl.ds(obase, _BINS)])
        pltpu.sync_copy(h_pred, out_hbm.at[pl.ds(obase + _BINS, _BINS)])
        pltpu.sync_copy(h_tp, out_hbm.at[pl.ds(obase + 2 * _BINS, _BINS)])

    return _sc_hist


def _f1_body(h_ref, out_ref):
    hs = jnp.sum(h_ref[...], axis=0)
    ct = hs[0:1, :]
    cp = hs[1:2, :]
    tp = hs[2:3, :]
    sens = tp / (cp + _EPS)
    prec = tp / (ct + _EPS)
    f1 = 2.0 * (prec * sens) / (prec + sens + _EPS)
    out_ref[0, 0] = jnp.sum(f1) / _CLASSES


@jax.jit
def kernel(y_pred, y_true):
    preds = pl.pallas_call(
        _argmax_body,
        grid=(_NUM_BLOCKS,),
        in_specs=[pl.BlockSpec((_ROWS_PER_BLOCK, _CLASSES), lambda i: (i, 0))],
        out_specs=pl.BlockSpec((_ROWS_PER_BLOCK,), lambda i: (i,)),
        out_shape=jax.ShapeDtypeStruct((_N,), jnp.int32),
    )(y_pred)
    partials = _make_sc_hist()(preds, y_true).reshape(_NUM_WORKERS, 3, _BINS)
    res = pl.pallas_call(
        _f1_body,
        out_shape=jax.ShapeDtypeStruct((1, 1), jnp.float32),
        out_specs=pl.BlockSpec(memory_space=pltpu.SMEM),
    )(partials)
    return res[0, 0]
